# baseline (speedup 1.0000x reference)
import jax
import jax.numpy as jnp
from jax.experimental import pallas as pl
from jax.experimental.pallas import tpu as pltpu

_CB = 128
_NBUF = 4


def _add_pos_kernel(x_hbm, pos_hbm, out_hbm, x_vmem, o_vmem, pos_vmem,
                    pos3_vmem, in_sems, out_sems, pos_sem):
    nb = x_hbm.shape[0]
    nc = nb // _CB
    s = x_hbm.shape[1]

    pltpu.make_async_copy(pos_hbm, pos_vmem, pos_sem).start()

    def in_copy(i, slot):
        return pltpu.make_async_copy(
            x_hbm.at[pl.ds(i * _CB, _CB)], x_vmem.at[slot], in_sems.at[slot])

    def out_copy(i, slot):
        return pltpu.make_async_copy(
            o_vmem.at[slot], out_hbm.at[pl.ds(i * _CB, _CB)], out_sems.at[slot])

    for k in range(min(_NBUF, nc)):
        in_copy(k, k).start(priority=0)

    pltpu.make_async_copy(pos_hbm, pos_vmem, pos_sem).wait()
    pos3_vmem[...] = jnp.broadcast_to(pos_vmem[:s, :][None], (_CB, s, x_hbm.shape[2]))

    for i in range(nc):
        slot = i % _NBUF
        in_copy(i, slot).wait()
        if i >= _NBUF:
            out_copy(i - _NBUF, slot).wait()
        o_vmem[slot] = x_vmem[slot] + pos3_vmem[...]
        out_copy(i, slot).start(priority=1)
        if i + _NBUF < nc:
            in_copy(i + _NBUF, slot).start(priority=0)

    for i in range(max(nc - _NBUF, 0), nc):
        out_copy(i, i % _NBUF).wait()


def kernel(concat_embeddings, pos_table):
    b, s, h = concat_embeddings.shape
    np_, _ = pos_table.shape
    return pl.pallas_call(
        _add_pos_kernel,
        in_specs=[
            pl.BlockSpec(memory_space=pltpu.MemorySpace.HBM),
            pl.BlockSpec(memory_space=pltpu.MemorySpace.HBM),
        ],
        out_specs=pl.BlockSpec(memory_space=pltpu.MemorySpace.HBM),
        out_shape=jax.ShapeDtypeStruct((b, s, h), concat_embeddings.dtype),
        scratch_shapes=[
            pltpu.VMEM((_NBUF, _CB, s, h), concat_embeddings.dtype),
            pltpu.VMEM((_NBUF, _CB, s, h), concat_embeddings.dtype),
            pltpu.VMEM((np_, h), pos_table.dtype),
            pltpu.VMEM((_CB, s, h), pos_table.dtype),
            pltpu.SemaphoreType.DMA((_NBUF,)),
            pltpu.SemaphoreType.DMA((_NBUF,)),
            pltpu.SemaphoreType.DMA,
        ],
    )(concat_embeddings, pos_table)
